# baseline (speedup 1.0000x reference)
import functools

import jax
import jax.numpy as jnp
from jax import lax
from jax.experimental import pallas as pl
from jax.experimental.pallas import tpu as pltpu
from jax.experimental.pallas import tpu_sc as plsc

N_IN = 100000
C = 128
N_OUT = 13000
M = 351000

NW = 32
SEG_PER_W = 408
LAST_SEGS = N_OUT - (NW - 1) * SEG_PER_W
CHUNK = 128
SUPER = 1024
SC_CHUNKS = SUPER // CHUNK
NEG_INF = float("-inf")


def _take_lane(vec, r):
    idx = jnp.full((16,), r, jnp.int32)
    dn = lax.GatherDimensionNumbers(
        offset_dims=(), collapsed_slice_dims=(0,), start_index_map=(0,))
    return lax.gather(vec, idx[:, None], dn, (1,),
                      mode=lax.GatherScatterMode.PROMISE_IN_BOUNDS)


def _lane0(vec):
    return lax.squeeze(lax.slice(vec, (0,), (1,)), (0,))


def _extract(meta_vecs, pos):
    lane = lax.iota(jnp.int32, 16)
    sel = jnp.zeros((16,), jnp.int32)
    for j, v in enumerate(meta_vecs):
        sel = sel | jnp.where(lane + (16 * j) == pos, v, 0)
    return _lane0(_take_lane(sel, lax.rem(pos, 16)))


def _sc_pool(feat_hbm, imap_hbm, omap_hbm, meta_hbm, out_hbm,
             meta_v, obig, idx_bufs, rows_bufs, slab_flat, gsems, isems):
    cid = lax.axis_index("c")
    sid = lax.axis_index("s")
    wid = sid * 2 + cid

    pltpu.sync_copy(meta_hbm, meta_v)
    meta_vecs = [meta_v[pl.ds(16 * j, 16)] for j in range(4)]
    start = _extract(meta_vecs, wid)
    end = _extract(meta_vecs, wid + NW)
    n = end - start
    nchunks = lax.div(n + (CHUNK - 1), CHUNK)
    nsc = lax.div(nchunks + (SC_CHUNKS - 1), SC_CHUNKS)

    seg_lo = pl.multiple_of(wid * SEG_PER_W, 8)
    seg_hi = jnp.minimum(seg_lo + SEG_PER_W, N_OUT)

    ninf16 = jnp.full((16,), NEG_INF, jnp.float32)

    def init_vec(i, _):
        slab_flat[pl.ds(pl.multiple_of(i * 16, 16), 16)] = ninf16
        return 0

    lax.fori_loop(0, SEG_PER_W * C // 16, init_vec, 0)

    lane = lax.iota(jnp.int32, 16)

    def flush(cur_vec, accs):
        owned = (cur_vec >= seg_lo) & (cur_vec < seg_hi)
        base = jnp.clip(cur_vec - seg_lo, 0, SEG_PER_W - 1) * C + lane
        for f in range(8):
            cur = plsc.load_gather(slab_flat, [base + 16 * f])
            plsc.store_scatter(slab_flat, [base + 16 * f],
                               jnp.maximum(cur, accs[f]), mask=owned)

    ninf16f = jnp.full((16,), NEG_INF, jnp.float32)
    empty_carry = (jnp.int32(-1), jnp.full((16,), -1, jnp.int32)) + \
        (ninf16f,) * 8

    def compute(j, u, carry):
        rows_u = rows_bufs[u]

        def do_group(g, carry):
            goff = pl.multiple_of(j * CHUNK + g * 16, 16)
            vec = obig[pl.ds(goff, 16)]
            kbase = g * 16
            s0 = _lane0(_take_lane(vec, 0))
            s15 = _lane0(_take_lane(vec, 15))

            def hom_path(carry):
                cur_s, cur_vec = carry[0], carry[1]
                accs = carry[2:]
                vals = [[rows_u[kbase + r, pl.ds(16 * f, 16)]
                         for f in range(8)] for r in range(16)]
                while len(vals) > 1:
                    vals = [[jnp.maximum(a[f], b[f]) for f in range(8)]
                            for a, b in zip(vals[::2], vals[1::2])]
                tree = vals[0]

                @pl.when(s0 != cur_s)
                def _():
                    flush(cur_vec, accs)

                same = vec == cur_vec
                new_accs = tuple(
                    jnp.where(same, jnp.maximum(accs[f], tree[f]), tree[f])
                    for f in range(8))
                return (s0, vec) + new_accs

            def mixed_path(carry):
                flush(carry[1], carry[2:])
                for r in range(16):
                    s_vec = _take_lane(vec, r)
                    owned = (s_vec >= seg_lo) & (s_vec < seg_hi)
                    base = (jnp.clip(s_vec - seg_lo, 0, SEG_PER_W - 1) * C
                            + lane)
                    rows = [rows_u[kbase + r, pl.ds(16 * f, 16)]
                            for f in range(8)]
                    curs = [plsc.load_gather(slab_flat, [base + 16 * f])
                            for f in range(8)]
                    for f in range(8):
                        plsc.store_scatter(slab_flat, [base + 16 * f],
                                           jnp.maximum(curs[f], rows[f]),
                                           mask=owned)
                return empty_carry

            return lax.cond(s0 == s15, hom_path, mixed_path, carry)

        return lax.fori_loop(0, SC_CHUNKS, do_group, carry)

    def wait_gather(u):
        pltpu.make_async_copy(
            feat_hbm.at[idx_bufs[u]], rows_bufs[u], gsems[u]).wait()

    def wait_idx(u):
        pltpu.make_async_copy(
            imap_hbm.at[pl.ds(0, CHUNK)], idx_bufs[u], isems[u]).wait()

    def coff(c):
        return pl.multiple_of(
            jnp.minimum(start + lax.div(c, SC_CHUNKS) * SUPER, M - SUPER)
            + lax.rem(c, SC_CHUNKS) * CHUNK, 8)

    def issue_idx(c, u):
        pltpu.async_copy(imap_hbm.at[pl.ds(coff(c), CHUNK)],
                         idx_bufs[u], isems[u])

    def issue_gather(u):
        pltpu.async_copy(feat_hbm.at[idx_bufs[u]], rows_bufs[u], gsems[u])

    o_last = jnp.minimum(
        jnp.maximum(start + (nsc - 1) * SUPER, 0), M - SUPER)
    t_last = jnp.clip(lax.div(end - o_last + (CHUNK - 1), CHUNK),
                      0, SC_CHUNKS)
    total = jnp.where(nsc == 0, 0, (nsc - 1) * SC_CHUNKS + t_last)
    npairs = lax.div(total + 1, 2)

    pltpu.sync_copy(imap_hbm.at[pl.ds(coff(0), CHUNK)], idx_bufs[0])
    pltpu.sync_copy(imap_hbm.at[pl.ds(coff(1), CHUNK)], idx_bufs[1])
    issue_gather(0)
    issue_gather(1)
    issue_idx(2, 2)
    issue_idx(3, 3)

    def do_pair(p, carry):
        c0 = p * 2

        def body(a, carry):
            bA, bB = 2 * a, 2 * a + 1
            bA2, bB2 = 2 - 2 * a, 3 - 2 * a

            @pl.when(lax.rem(c0, SC_CHUNKS) == 0)
            def _():
                pltpu.sync_copy(omap_hbm.at[pl.ds(coff(c0), SUPER)],
                                obig)

            jl = lax.rem(c0, SC_CHUNKS)
            wait_idx(bA2)
            issue_gather(bA2)
            wait_idx(bB2)
            issue_gather(bB2)
            wait_gather(bA)
            issue_idx(c0 + 4, bA)
            carry = compute(jl, bA, carry)
            wait_gather(bB)
            issue_idx(c0 + 5, bB)
            return compute(jl + 1, bB, carry)

        return lax.cond(lax.rem(p, 2) == 0,
                        lambda cr: body(0, cr),
                        lambda cr: body(1, cr), carry)

    carry = lax.fori_loop(0, npairs, do_pair, empty_carry)

    @pl.when(lax.rem(npairs, 2) == 0)
    def _():
        wait_gather(0)
        wait_gather(1)
        wait_idx(2)
        wait_idx(3)

    @pl.when(lax.rem(npairs, 2) == 1)
    def _():
        wait_gather(2)
        wait_gather(3)
        wait_idx(0)
        wait_idx(1)

    flush(carry[1], carry[2:])

    def fix_vec(i, _):
        off = pl.multiple_of(i * 16, 16)
        v = slab_flat[pl.ds(off, 16)]
        slab_flat[pl.ds(off, 16)] = jnp.where(v == NEG_INF, 0.0, v)
        return 0

    lax.fori_loop(0, SEG_PER_W * C // 16, fix_vec, 0)

    out_off = pl.multiple_of(seg_lo * C, 8)

    @pl.when(wid < NW - 1)
    def _():
        pltpu.sync_copy(slab_flat,
                        out_hbm.at[pl.ds(out_off, SEG_PER_W * C)])

    @pl.when(wid == NW - 1)
    def _():
        pltpu.sync_copy(slab_flat.at[pl.ds(0, LAST_SEGS * C)],
                        out_hbm.at[pl.ds(out_off, LAST_SEGS * C)])


def _sc_pool_entry(feat_hbm, imap_hbm, omap_hbm, meta_hbm, out_hbm,
                   meta_v, obig, i0, i1, i2, i3, r0, r1, r2, r3,
                   slab_flat, g0, g1, g2, g3, s0, s1, s2, s3):
    _sc_pool(feat_hbm, imap_hbm, omap_hbm, meta_hbm, out_hbm,
             meta_v, obig, (i0, i1, i2, i3), (r0, r1, r2, r3),
             slab_flat, (g0, g1, g2, g3), (s0, s1, s2, s3))


@jax.jit
def kernel(input_features, in_map, out_map):
    in_map = in_map.astype(jnp.int32)
    out_map = out_map.astype(jnp.int32)

    targets = jnp.arange(1, NW, dtype=jnp.int32) * SEG_PER_W
    inner = jnp.searchsorted(out_map, targets, side="left").astype(jnp.int32)
    bounds = jnp.concatenate(
        [jnp.zeros((1,), jnp.int32), inner, jnp.full((1,), M, jnp.int32)])
    starts8 = (bounds[:NW] // 8) * 8
    ends = bounds[1:]
    meta = jnp.concatenate([starts8, ends])

    mesh = plsc.VectorSubcoreMesh(core_axis_name="c", subcore_axis_name="s")
    f = functools.partial(
        pl.kernel,
        mesh=mesh,
        compiler_params=pltpu.CompilerParams(needs_layout_passes=False),
        out_type=jax.ShapeDtypeStruct((N_OUT * C,), jnp.float32),
        scratch_types=[
            pltpu.VMEM((64,), jnp.int32),
            pltpu.VMEM((SUPER,), jnp.int32),
            *[pltpu.VMEM((CHUNK,), jnp.int32) for _ in range(4)],
            *[pltpu.VMEM((CHUNK, C), jnp.float32) for _ in range(4)],
            pltpu.VMEM((SEG_PER_W * C,), jnp.float32),
            *[pltpu.SemaphoreType.DMA for _ in range(8)],
        ],
    )(_sc_pool_entry)
    return f(input_features, in_map, out_map, meta).reshape(N_OUT, C)


# speedup vs baseline: 1.2435x; 1.2435x over previous
import functools

import jax
import jax.numpy as jnp
from jax import lax
from jax.experimental import pallas as pl
from jax.experimental.pallas import tpu as pltpu
from jax.experimental.pallas import tpu_sc as plsc

N_IN = 100000
C = 128
N_OUT = 13000
M = 351000

NW = 32
SEG_PER_W = 408
LAST_SEGS = N_OUT - (NW - 1) * SEG_PER_W
CHUNK = 128
SUPER = 1024
SC_CHUNKS = SUPER // CHUNK
NEG_INF = float("-inf")


def _take_lane(vec, r):
    idx = jnp.full((16,), r, jnp.int32)
    dn = lax.GatherDimensionNumbers(
        offset_dims=(), collapsed_slice_dims=(0,), start_index_map=(0,))
    return lax.gather(vec, idx[:, None], dn, (1,),
                      mode=lax.GatherScatterMode.PROMISE_IN_BOUNDS)


def _lane0(vec):
    return lax.squeeze(lax.slice(vec, (0,), (1,)), (0,))


def _extract(meta_vecs, pos):
    lane = lax.iota(jnp.int32, 16)
    sel = jnp.zeros((16,), jnp.int32)
    for j, v in enumerate(meta_vecs):
        sel = sel | jnp.where(lane + (16 * j) == pos, v, 0)
    return _lane0(_take_lane(sel, lax.rem(pos, 16)))


def _sc_pool(feat_hbm, imap_hbm, omap_hbm, meta_hbm, out_hbm,
             meta_v, obig, idx_bufs, rows_bufs, slab_flat, gsems, isems):
    cid = lax.axis_index("c")
    sid = lax.axis_index("s")
    wid = sid * 2 + cid

    pltpu.sync_copy(meta_hbm, meta_v)
    meta_vecs = [meta_v[pl.ds(16 * j, 16)] for j in range(4)]
    start = _extract(meta_vecs, wid)
    end = _extract(meta_vecs, wid + NW)
    n = end - start
    nchunks = lax.div(n + (CHUNK - 1), CHUNK)
    nsc = lax.div(nchunks + (SC_CHUNKS - 1), SC_CHUNKS)

    seg_lo = pl.multiple_of(wid * SEG_PER_W, 8)
    seg_hi = jnp.minimum(seg_lo + SEG_PER_W, N_OUT)

    ninf16 = jnp.full((16,), NEG_INF, jnp.float32)

    def init_vec(i, _):
        slab_flat[pl.ds(pl.multiple_of(i * 16, 16), 16)] = ninf16
        return 0

    lax.fori_loop(0, SEG_PER_W * C // 16, init_vec, 0)

    lane = lax.iota(jnp.int32, 16)

    def flush(cur_vec, accs):
        owned = (cur_vec >= seg_lo) & (cur_vec < seg_hi)
        base = jnp.clip(cur_vec - seg_lo, 0, SEG_PER_W - 1) * C + lane
        for f in range(8):
            cur = plsc.load_gather(slab_flat, [base + 16 * f])
            plsc.store_scatter(slab_flat, [base + 16 * f],
                               jnp.maximum(cur, accs[f]), mask=owned)

    ninf16f = jnp.full((16,), NEG_INF, jnp.float32)
    empty_carry = (jnp.int32(-1), jnp.full((16,), -1, jnp.int32)) + \
        (ninf16f,) * 8

    def compute(j, u, carry):
        rows_u = rows_bufs[u]

        def do_group(g, carry):
            goff = pl.multiple_of(j * CHUNK + g * 16, 16)
            vec = obig[pl.ds(goff, 16)]
            kbase = g * 16
            s0 = _lane0(_take_lane(vec, 0))
            s15 = _lane0(_take_lane(vec, 15))

            def hom_path(carry):
                cur_s, cur_vec = carry[0], carry[1]
                accs = carry[2:]
                vals = [[rows_u[kbase + r, pl.ds(16 * f, 16)]
                         for f in range(8)] for r in range(16)]
                while len(vals) > 1:
                    vals = [[jnp.maximum(a[f], b[f]) for f in range(8)]
                            for a, b in zip(vals[::2], vals[1::2])]
                tree = vals[0]

                @pl.when(s0 != cur_s)
                def _():
                    flush(cur_vec, accs)

                same = vec == cur_vec
                new_accs = tuple(
                    jnp.where(same, jnp.maximum(accs[f], tree[f]), tree[f])
                    for f in range(8))
                return (s0, vec) + new_accs

            def mixed_path(carry):
                flush(carry[1], carry[2:])
                for r in range(16):
                    s_vec = _take_lane(vec, r)
                    owned = (s_vec >= seg_lo) & (s_vec < seg_hi)
                    base = (jnp.clip(s_vec - seg_lo, 0, SEG_PER_W - 1) * C
                            + lane)
                    rows = [rows_u[kbase + r, pl.ds(16 * f, 16)]
                            for f in range(8)]
                    curs = [plsc.load_gather(slab_flat, [base + 16 * f])
                            for f in range(8)]
                    for f in range(8):
                        plsc.store_scatter(slab_flat, [base + 16 * f],
                                           jnp.maximum(curs[f], rows[f]),
                                           mask=owned)
                return empty_carry

            return lax.cond(s0 == s15, hom_path, mixed_path, carry)

        return lax.fori_loop(0, SC_CHUNKS, do_group, carry)

    def wait_gather(u):
        pltpu.make_async_copy(
            feat_hbm.at[idx_bufs[u]], rows_bufs[u], gsems[u]).wait()

    def wait_idx(u):
        pltpu.make_async_copy(
            imap_hbm.at[pl.ds(0, CHUNK)], idx_bufs[u], isems[u]).wait()

    def coff(c):
        return pl.multiple_of(
            jnp.minimum(start + lax.div(c, SC_CHUNKS) * SUPER, M - SUPER)
            + lax.rem(c, SC_CHUNKS) * CHUNK, 8)

    def issue_idx(c, u):
        pltpu.async_copy(imap_hbm.at[pl.ds(coff(c), CHUNK)],
                         idx_bufs[u], isems[u])

    def issue_gather(u):
        pltpu.async_copy(feat_hbm.at[idx_bufs[u]], rows_bufs[u], gsems[u])

    o_last = jnp.minimum(
        jnp.maximum(start + (nsc - 1) * SUPER, 0), M - SUPER)
    t_last = jnp.clip(lax.div(end - o_last + (CHUNK - 1), CHUNK),
                      0, SC_CHUNKS)
    total = jnp.where(nsc == 0, 0, (nsc - 1) * SC_CHUNKS + t_last)
    pltpu.sync_copy(imap_hbm.at[pl.ds(coff(0), CHUNK)], idx_bufs[0])
    pltpu.sync_copy(imap_hbm.at[pl.ds(coff(1), CHUNK)], idx_bufs[1])
    issue_gather(0)
    issue_gather(1)

    def do_chunk(j, carry):
        def body(u, carry):
            wait_gather(u)
            issue_idx(j + 2, u)

            @pl.when(lax.rem(j, SC_CHUNKS) == 0)
            def _():
                pltpu.sync_copy(omap_hbm.at[pl.ds(coff(j), SUPER)],
                                obig)

            carry = compute(lax.rem(j, SC_CHUNKS), u, carry)
            wait_idx(u)
            issue_gather(u)
            return carry

        return lax.cond(lax.rem(j, 2) == 0,
                        lambda cr: body(0, cr),
                        lambda cr: body(1, cr), carry)

    carry = lax.fori_loop(0, total, do_chunk, empty_carry)

    wait_gather(0)
    wait_gather(1)

    flush(carry[1], carry[2:])

    def fix_vec(i, _):
        off = pl.multiple_of(i * 16, 16)
        v = slab_flat[pl.ds(off, 16)]
        slab_flat[pl.ds(off, 16)] = jnp.where(v == NEG_INF, 0.0, v)
        return 0

    lax.fori_loop(0, SEG_PER_W * C // 16, fix_vec, 0)

    out_off = pl.multiple_of(seg_lo * C, 8)

    @pl.when(wid < NW - 1)
    def _():
        pltpu.sync_copy(slab_flat,
                        out_hbm.at[pl.ds(out_off, SEG_PER_W * C)])

    @pl.when(wid == NW - 1)
    def _():
        pltpu.sync_copy(slab_flat.at[pl.ds(0, LAST_SEGS * C)],
                        out_hbm.at[pl.ds(out_off, LAST_SEGS * C)])


def _sc_pool_entry(feat_hbm, imap_hbm, omap_hbm, meta_hbm, out_hbm,
                   meta_v, obig, i0, i1, r0, r1,
                   slab_flat, g0, g1, s0, s1):
    _sc_pool(feat_hbm, imap_hbm, omap_hbm, meta_hbm, out_hbm,
             meta_v, obig, (i0, i1), (r0, r1),
             slab_flat, (g0, g1), (s0, s1))


@jax.jit
def kernel(input_features, in_map, out_map):
    in_map = in_map.astype(jnp.int32)
    out_map = out_map.astype(jnp.int32)

    targets = jnp.arange(1, NW, dtype=jnp.int32) * SEG_PER_W
    inner = jnp.searchsorted(out_map, targets, side="left").astype(jnp.int32)
    bounds = jnp.concatenate(
        [jnp.zeros((1,), jnp.int32), inner, jnp.full((1,), M, jnp.int32)])
    starts8 = (bounds[:NW] // 8) * 8
    ends = bounds[1:]
    meta = jnp.concatenate([starts8, ends])

    mesh = plsc.VectorSubcoreMesh(core_axis_name="c", subcore_axis_name="s")
    f = functools.partial(
        pl.kernel,
        mesh=mesh,
        compiler_params=pltpu.CompilerParams(needs_layout_passes=False),
        out_type=jax.ShapeDtypeStruct((N_OUT * C,), jnp.float32),
        scratch_types=[
            pltpu.VMEM((64,), jnp.int32),
            pltpu.VMEM((SUPER,), jnp.int32),
            *[pltpu.VMEM((CHUNK,), jnp.int32) for _ in range(2)],
            *[pltpu.VMEM((CHUNK, C), jnp.float32) for _ in range(2)],
            pltpu.VMEM((SEG_PER_W * C,), jnp.float32),
            *[pltpu.SemaphoreType.DMA for _ in range(4)],
        ],
    )(_sc_pool_entry)
    return f(input_features, in_map, out_map, meta).reshape(N_OUT, C)
